# baseline (speedup 1.0000x reference)
import functools

import jax
import jax.numpy as jnp
from jax import lax
from jax.experimental import pallas as pl
from jax.experimental.pallas import tpu as pltpu
from jax.experimental.pallas import tpu_sc as plsc

N = 10000
E = 320000
D = 128
NEXP = 4

NTILES = 32
CHUNK = 80
NCHUNKS = 4096
EPT = NCHUNKS // NTILES
NPAD = 10240
ROWS_PER_TILE = NPAD // 16
ZROWS = 128
CNTW = 16
EPAD = NCHUNKS * CHUNK


def _sc_segsum_body(y_hbm, src_hbm, dst_hbm, s0_out, s1_out,
                    src_v, dst_v, rows_a, rows_b, sums_sh, sem_a, sem_b):
    cid = lax.axis_index("c")
    sid = lax.axis_index("s")
    wid = cid * 16 + sid
    r0 = sid * ROWS_PER_TILE
    zrow = jnp.zeros((16,), jnp.float32)

    def fill_z(i, _):
        for c in range(8):
            rows_a[i, pl.ds(c * 16, 16)] = zrow
        return _

    lax.fori_loop(0, CHUNK, fill_z, None)
    for cp in range(ROWS_PER_TILE // CHUNK):
        pltpu.sync_copy(rows_a, sums_sh.at[pl.ds(r0 + cp * CHUNK, CHUNK)])
    plsc.subcore_barrier()

    pltpu.sync_copy(src_hbm.at[pl.ds(wid * EPT, EPT)], src_v)
    pltpu.sync_copy(dst_hbm.at[pl.ds(wid * EPT, EPT)], dst_v)

    def chunk_body(j, _):
        pltpu.async_copy(y_hbm.at[src_v.at[j]], rows_a, sem_a).wait()
        pltpu.sync_copy(rows_a, sums_sh.at[dst_v.at[j]], add=True)
        return _

    lax.fori_loop(0, EPT, chunk_body, None)

    plsc.subcore_barrier()

    @pl.when(cid == 0)
    def _():
        pltpu.sync_copy(sums_sh.at[pl.ds(r0, ROWS_PER_TILE)],
                        s0_out.at[pl.ds(r0, ROWS_PER_TILE)])

    @pl.when(cid == 1)
    def _():
        pltpu.sync_copy(sums_sh.at[pl.ds(r0, ROWS_PER_TILE)],
                        s1_out.at[pl.ds(r0, ROWS_PER_TILE)])


_sc_segsum = functools.partial(
    pl.kernel,
    out_type=[
        jax.ShapeDtypeStruct((NPAD, D), jnp.float32),
        jax.ShapeDtypeStruct((NPAD, D), jnp.float32),
    ],
    mesh=plsc.VectorSubcoreMesh(core_axis_name="c", subcore_axis_name="s"),
    scratch_types=[
        pltpu.VMEM((EPT, CHUNK), jnp.int32),
        pltpu.VMEM((EPT, CHUNK), jnp.int32),
        pltpu.VMEM((CHUNK, D), jnp.float32),
        pltpu.VMEM((CHUNK, D), jnp.float32),
        pltpu.VMEM_SHARED((NPAD, D), jnp.float32),
        pltpu.SemaphoreType.DMA,
        pltpu.SemaphoreType.DMA,
    ],
)(_sc_segsum_body)


def _sc_cnt_body(dst_hbm, c0_out, c1_out, dst_v, ones_v, zb16, cnt_sh, sem):
    cid = lax.axis_index("c")
    sid = lax.axis_index("s")
    wid = cid * 16 + sid
    r0 = sid * ROWS_PER_TILE
    zrow = jnp.zeros((16,), jnp.float32)

    pltpu.sync_copy(dst_hbm.at[pl.ds(wid * EPT, EPT)], dst_v)

    def fill_ones(i, _):
        ones_v[i, :] = zrow + 1.0
        return _

    lax.fori_loop(0, CHUNK, fill_ones, None)

    def fill_z(i, _):
        zb16[i, :] = zrow
        return _

    lax.fori_loop(0, ZROWS, fill_z, None)
    for cp in range(ROWS_PER_TILE // ZROWS):
        pltpu.sync_copy(zb16, cnt_sh.at[pl.ds(r0 + cp * ZROWS, ZROWS)])
    plsc.subcore_barrier()

    def chunk_body(j, _):
        pltpu.sync_copy(ones_v, cnt_sh.at[dst_v.at[j]], add=True)
        return _

    lax.fori_loop(0, EPT, chunk_body, None)
    plsc.subcore_barrier()

    @pl.when(cid == 0)
    def _():
        pltpu.sync_copy(cnt_sh.at[pl.ds(r0, ROWS_PER_TILE)],
                        c0_out.at[pl.ds(r0, ROWS_PER_TILE)])

    @pl.when(cid == 1)
    def _():
        pltpu.sync_copy(cnt_sh.at[pl.ds(r0, ROWS_PER_TILE)],
                        c1_out.at[pl.ds(r0, ROWS_PER_TILE)])


_sc_cnt = functools.partial(
    pl.kernel,
    out_type=[
        jax.ShapeDtypeStruct((NPAD, CNTW), jnp.float32),
        jax.ShapeDtypeStruct((NPAD, CNTW), jnp.float32),
    ],
    mesh=plsc.VectorSubcoreMesh(core_axis_name="c", subcore_axis_name="s"),
    scratch_types=[
        pltpu.VMEM((EPT, CHUNK), jnp.int32),
        pltpu.VMEM((CHUNK, CNTW), jnp.float32),
        pltpu.VMEM((ZROWS, CNTW), jnp.float32),
        pltpu.VMEM_SHARED((NPAD, CNTW), jnp.float32),
        pltpu.SemaphoreType.DMA,
    ],
)(_sc_cnt_body)


BLK = 1000


def _prelin_body(x_ref, w_ref, b_ref, o_ref):
    o_ref[...] = jnp.maximum(
        jnp.dot(x_ref[...], w_ref[...], preferred_element_type=jnp.float32)
        + b_ref[...], 0.0)


def _prelin(x, w, b):
    n, d = x.shape
    h = w.shape[1]
    return pl.pallas_call(
        _prelin_body,
        grid=(n // BLK,),
        in_specs=[
            pl.BlockSpec((BLK, d), lambda i: (i, 0)),
            pl.BlockSpec((d, h), lambda i: (0, 0)),
            pl.BlockSpec((1, h), lambda i: (0, 0)),
        ],
        out_specs=pl.BlockSpec((BLK, h), lambda i: (i, 0)),
        out_shape=jax.ShapeDtypeStruct((n, h), jnp.float32),
    )(x, w, b.reshape(1, h))


def _moe_common(x_ref, s0_ref, s1_ref, c0_ref, c1_ref, wgx_ref, wga_ref,
                wgb_ref, awx_ref, awa_ref, ab_ref):
    x = x_ref[...]
    cnt = c0_ref[:, 0:1] + c1_ref[:, 0:1]
    aggr = (s0_ref[...] + s1_ref[...]) / jnp.maximum(cnt, 1.0)
    logits = (jnp.dot(x, wgx_ref[...], preferred_element_type=jnp.float32)
              + jnp.dot(aggr, wga_ref[...], preferred_element_type=jnp.float32)
              + wgb_ref[...])
    m = jnp.max(logits, axis=1, keepdims=True)
    eg = jnp.exp(logits - m)
    g = eg / jnp.sum(eg, axis=1, keepdims=True)
    out = jnp.zeros_like(aggr)
    for i in range(NEXP):
        ei = jnp.maximum(
            jnp.dot(x, awx_ref[i], preferred_element_type=jnp.float32)
            + jnp.dot(aggr, awa_ref[i], preferred_element_type=jnp.float32)
            + ab_ref[:, i, :], 0.0)
        out = out + ei * g[:, i:i + 1]
    nrm = jnp.maximum(jnp.sqrt(jnp.sum(out * out, axis=1, keepdims=True)),
                      1e-12)
    return jnp.maximum(out / nrm, 0.0)


def _moe_mid_body(x_ref, s0_ref, s1_ref, c0_ref, c1_ref, wgx_ref, wga_ref,
                  wgb_ref, awx_ref, awa_ref, ab_ref, lw_ref, lb_ref,
                  h_ref, y_ref):
    h = _moe_common(x_ref, s0_ref, s1_ref, c0_ref, c1_ref, wgx_ref, wga_ref,
                    wgb_ref, awx_ref, awa_ref, ab_ref)
    h_ref[...] = h
    y_ref[...] = jnp.maximum(
        jnp.dot(h, lw_ref[...], preferred_element_type=jnp.float32)
        + lb_ref[...], 0.0)


def _moe_fin_body(x_ref, s0_ref, s1_ref, c0_ref, c1_ref, wgx_ref, wga_ref,
                  wgb_ref, awx_ref, awa_ref, ab_ref, p1w_ref, p1b_ref,
                  p2w_ref, p2b_ref, o_ref):
    h = _moe_common(x_ref, s0_ref, s1_ref, c0_ref, c1_ref, wgx_ref, wga_ref,
                    wgb_ref, awx_ref, awa_ref, ab_ref)
    z = (jnp.dot(h, p1w_ref[...], preferred_element_type=jnp.float32)
         + p1b_ref[...])
    z = (jnp.dot(z, p2w_ref[...], preferred_element_type=jnp.float32)
         + p2b_ref[...])
    m = jnp.max(z, axis=1, keepdims=True)
    lse = m + jnp.log(jnp.sum(jnp.exp(z - m), axis=1, keepdims=True))
    o_ref[...] = z - lse


def _moe_specs(din, h, extra_specs):
    row = lambda w: pl.BlockSpec((BLK, w), lambda i: (i, 0))
    full = lambda *s: pl.BlockSpec(s, lambda i: (0,) * len(s))
    return [
        row(din), row(h), row(h), row(CNTW), row(CNTW),
        full(din, NEXP), full(h, NEXP), full(1, NEXP),
        full(NEXP, din, h), full(NEXP, h, h), full(1, NEXP, h),
    ] + extra_specs


def _moe_mid(x, s0, s1, c0, c1, wg_w, wg_b, agg_w, agg_b, lw, lb):
    n, din = x.shape
    h = s0.shape[1]
    full = lambda *s: pl.BlockSpec(s, lambda i: (0,) * len(s))
    return pl.pallas_call(
        _moe_mid_body,
        grid=(n // BLK,),
        in_specs=_moe_specs(din, h, [full(h, h), full(1, h)]),
        out_specs=[pl.BlockSpec((BLK, h), lambda i: (i, 0)),
                   pl.BlockSpec((BLK, h), lambda i: (i, 0))],
        out_shape=[jax.ShapeDtypeStruct((n, h), jnp.float32),
                   jax.ShapeDtypeStruct((n, h), jnp.float32)],
    )(x, s0, s1, c0, c1, wg_w[:din], wg_w[din:], wg_b.reshape(1, NEXP),
      agg_w[:, :din], agg_w[:, din:], agg_b.reshape(1, NEXP, h),
      lw, lb.reshape(1, h))


def _moe_fin(x, s0, s1, c0, c1, wg_w, wg_b, agg_w, agg_b, p1w, p1b, p2w, p2b):
    n, din = x.shape
    h = s0.shape[1]
    o = p2w.shape[1]
    full = lambda *s: pl.BlockSpec(s, lambda i: (0,) * len(s))
    return pl.pallas_call(
        _moe_fin_body,
        grid=(n // BLK,),
        in_specs=_moe_specs(din, h, [full(h, h), full(1, h),
                                     full(h, o), full(1, o)]),
        out_specs=pl.BlockSpec((BLK, o), lambda i: (i, 0)),
        out_shape=jax.ShapeDtypeStruct((n, o), jnp.float32),
    )(x, s0, s1, c0, c1, wg_w[:din], wg_w[din:], wg_b.reshape(1, NEXP),
      agg_w[:, :din], agg_w[:, din:], agg_b.reshape(1, NEXP, h),
      p1w, p1b.reshape(1, h), p2w, p2b.reshape(1, o))


def kernel(x, edge_index, lin_W0, lin_b0, Wg_W0, Wg_b0, agg_W0, agg_b0,
           lin_W1, lin_b1, Wg_W1, Wg_b1, agg_W1, agg_b1,
           p1_W, p1_b, p2_W, p2_b):
    src = edge_index[0]
    dst = edge_index[1]
    npad_edges = EPAD - src.shape[0]
    srcp = jnp.concatenate(
        [src, jnp.zeros((npad_edges,), jnp.int32)]).reshape(NCHUNKS, CHUNK)
    pad_dst = N + jnp.arange(npad_edges, dtype=jnp.int32) % (NPAD - N)
    dstp = jnp.concatenate([dst, pad_dst]).reshape(NCHUNKS, CHUNK)

    y0 = _prelin(x, lin_W0, lin_b0)
    c0a, c0b = _sc_cnt(dstp)
    s0a, s0b = _sc_segsum(y0, srcp, dstp)
    h1, y1 = _moe_mid(x, s0a, s0b, c0a, c0b, Wg_W0, Wg_b0, agg_W0, agg_b0,
                      lin_W1, lin_b1)
    s1a, s1b = _sc_segsum(y1, srcp, dstp)
    return _moe_fin(h1, s1a, s1b, c0a, c0b, Wg_W1, Wg_b1, agg_W1, agg_b1,
                    p1_W, p1_b, p2_W, p2_b)


# speedup vs baseline: 1.1464x; 1.1464x over previous
import functools

import jax
import jax.numpy as jnp
from jax import lax
from jax.experimental import pallas as pl
from jax.experimental.pallas import tpu as pltpu
from jax.experimental.pallas import tpu_sc as plsc

N = 10000
E = 320000
D = 128
NEXP = 4

NTILES = 32
CHUNK = 128
NCHUNKS = 2560
EPT = NCHUNKS // NTILES
NPAD = 10112
ROWS_PER_TILE = NPAD // 16
EPT0 = 32
EPT1 = 128
CNTW = 16
EPAD = NCHUNKS * CHUNK


def _sc_segsum_body(y_hbm, src_hbm, dst_hbm, s0_out, s1_out,
                    src_v, dst_v, rows_a, rows_b, sums_sh, sem_a, sem_b):
    cid = lax.axis_index("c")
    sid = lax.axis_index("s")
    r0 = sid * ROWS_PER_TILE
    zrow = jnp.zeros((16,), jnp.float32)

    def fill_z(i, _):
        for c in range(8):
            rows_a[i, pl.ds(c * 16, 16)] = zrow
        return _

    lax.fori_loop(0, CHUNK, fill_z, None)
    for cp in range(ROWS_PER_TILE // CHUNK):
        pltpu.sync_copy(rows_a, sums_sh.at[pl.ds(r0 + cp * CHUNK, CHUNK)])
    pltpu.sync_copy(rows_a.at[pl.ds(0, ROWS_PER_TILE % CHUNK)],
                    sums_sh.at[pl.ds(r0 + (ROWS_PER_TILE // CHUNK) * CHUNK,
                                     ROWS_PER_TILE % CHUNK)])
    plsc.subcore_barrier()

    def run_pairs(npairs):
        pltpu.async_copy(y_hbm.at[src_v.at[0]], rows_a, sem_a)

        def pair_body(p, _):
            j0 = 2 * p
            j1 = j0 + 1
            pltpu.async_copy(y_hbm.at[src_v.at[j1]], rows_b, sem_b)
            pltpu.make_async_copy(y_hbm.at[src_v.at[j0]], rows_a,
                                  sem_a).wait()
            pltpu.sync_copy(rows_a, sums_sh.at[dst_v.at[j0]], add=True)

            @pl.when(p < npairs - 1)
            def _():
                pltpu.async_copy(y_hbm.at[src_v.at[j0 + 2]], rows_a, sem_a)

            pltpu.make_async_copy(y_hbm.at[src_v.at[j1]], rows_b,
                                  sem_b).wait()
            pltpu.sync_copy(rows_b, sums_sh.at[dst_v.at[j1]], add=True)
            return _

        lax.fori_loop(0, npairs, pair_body, None)

    @pl.when(cid == 0)
    def _():
        rb = sid * EPT0
        pltpu.sync_copy(src_hbm.at[pl.ds(rb, EPT0)],
                        src_v.at[pl.ds(0, EPT0)])
        pltpu.sync_copy(dst_hbm.at[pl.ds(rb, EPT0)],
                        dst_v.at[pl.ds(0, EPT0)])
        run_pairs(EPT0 // 2)

    @pl.when(cid == 1)
    def _():
        for half in range(2):
            rb = 16 * EPT0 + sid * EPT1 + half * (EPT1 // 2)
            pltpu.sync_copy(src_hbm.at[pl.ds(rb, EPT1 // 2)], src_v)
            pltpu.sync_copy(dst_hbm.at[pl.ds(rb, EPT1 // 2)], dst_v)
            run_pairs(EPT1 // 4)

    plsc.subcore_barrier()

    @pl.when(cid == 0)
    def _():
        pltpu.sync_copy(sums_sh.at[pl.ds(r0, ROWS_PER_TILE)],
                        s0_out.at[pl.ds(r0, ROWS_PER_TILE)])

    @pl.when(cid == 1)
    def _():
        pltpu.sync_copy(sums_sh.at[pl.ds(r0, ROWS_PER_TILE)],
                        s1_out.at[pl.ds(r0, ROWS_PER_TILE)])


_sc_segsum = functools.partial(
    pl.kernel,
    out_type=[
        jax.ShapeDtypeStruct((NPAD, D), jnp.float32),
        jax.ShapeDtypeStruct((NPAD, D), jnp.float32),
    ],
    mesh=plsc.VectorSubcoreMesh(core_axis_name="c", subcore_axis_name="s"),
    scratch_types=[
        pltpu.VMEM((EPT1 // 2, CHUNK), jnp.int32),
        pltpu.VMEM((EPT1 // 2, CHUNK), jnp.int32),
        pltpu.VMEM((CHUNK, D), jnp.float32),
        pltpu.VMEM((CHUNK, D), jnp.float32),
        pltpu.VMEM_SHARED((NPAD, D), jnp.float32),
        pltpu.SemaphoreType.DMA,
        pltpu.SemaphoreType.DMA,
    ],
)(_sc_segsum_body)


def _sc_cnt_body(dst_hbm, c0_out, c1_out, dst_v, ones_v, zb16, cnt_sh, sem):
    cid = lax.axis_index("c")
    sid = lax.axis_index("s")
    wid = cid * 16 + sid
    r0 = sid * ROWS_PER_TILE
    zrow = jnp.zeros((16,), jnp.float32)

    pltpu.sync_copy(dst_hbm.at[pl.ds(wid * EPT, EPT)], dst_v)

    def fill_ones(i, _):
        ones_v[i, :] = zrow + 1.0
        zb16[i, :] = zrow
        return _

    lax.fori_loop(0, CHUNK, fill_ones, None)
    for cp in range(ROWS_PER_TILE // CHUNK):
        pltpu.sync_copy(zb16, cnt_sh.at[pl.ds(r0 + cp * CHUNK, CHUNK)])
    pltpu.sync_copy(zb16.at[pl.ds(0, ROWS_PER_TILE % CHUNK)],
                    cnt_sh.at[pl.ds(r0 + (ROWS_PER_TILE // CHUNK) * CHUNK,
                                    ROWS_PER_TILE % CHUNK)])
    plsc.subcore_barrier()

    def chunk_body(j, _):
        pltpu.sync_copy(ones_v, cnt_sh.at[dst_v.at[j]], add=True)
        return _

    lax.fori_loop(0, EPT, chunk_body, None)
    plsc.subcore_barrier()

    @pl.when(cid == 0)
    def _():
        pltpu.sync_copy(cnt_sh.at[pl.ds(r0, ROWS_PER_TILE)],
                        c0_out.at[pl.ds(r0, ROWS_PER_TILE)])

    @pl.when(cid == 1)
    def _():
        pltpu.sync_copy(cnt_sh.at[pl.ds(r0, ROWS_PER_TILE)],
                        c1_out.at[pl.ds(r0, ROWS_PER_TILE)])


_sc_cnt = functools.partial(
    pl.kernel,
    out_type=[
        jax.ShapeDtypeStruct((NPAD, CNTW), jnp.float32),
        jax.ShapeDtypeStruct((NPAD, CNTW), jnp.float32),
    ],
    mesh=plsc.VectorSubcoreMesh(core_axis_name="c", subcore_axis_name="s"),
    scratch_types=[
        pltpu.VMEM((EPT, CHUNK), jnp.int32),
        pltpu.VMEM((CHUNK, CNTW), jnp.float32),
        pltpu.VMEM((CHUNK, CNTW), jnp.float32),
        pltpu.VMEM_SHARED((NPAD, CNTW), jnp.float32),
        pltpu.SemaphoreType.DMA,
    ],
)(_sc_cnt_body)


BLK = 1000


def _prelin_body(x_ref, w_ref, b_ref, o_ref):
    o_ref[...] = jnp.maximum(
        jnp.dot(x_ref[...], w_ref[...], preferred_element_type=jnp.float32)
        + b_ref[...], 0.0)


def _prelin(x, w, b):
    n, d = x.shape
    h = w.shape[1]
    return pl.pallas_call(
        _prelin_body,
        grid=(n // BLK,),
        in_specs=[
            pl.BlockSpec((BLK, d), lambda i: (i, 0)),
            pl.BlockSpec((d, h), lambda i: (0, 0)),
            pl.BlockSpec((1, h), lambda i: (0, 0)),
        ],
        out_specs=pl.BlockSpec((BLK, h), lambda i: (i, 0)),
        out_shape=jax.ShapeDtypeStruct((n, h), jnp.float32),
    )(x, w, b.reshape(1, h))


def _moe_common(x_ref, s0_ref, s1_ref, c0_ref, c1_ref, wgx_ref, wga_ref,
                wgb_ref, awx_ref, awa_ref, ab_ref):
    x = x_ref[...]
    cnt = c0_ref[:, 0:1] + c1_ref[:, 0:1]
    aggr = (s0_ref[...] + s1_ref[...]) / jnp.maximum(cnt, 1.0)
    logits = (jnp.dot(x, wgx_ref[...], preferred_element_type=jnp.float32)
              + jnp.dot(aggr, wga_ref[...], preferred_element_type=jnp.float32)
              + wgb_ref[...])
    m = jnp.max(logits, axis=1, keepdims=True)
    eg = jnp.exp(logits - m)
    g = eg / jnp.sum(eg, axis=1, keepdims=True)
    out = jnp.zeros_like(aggr)
    for i in range(NEXP):
        ei = jnp.maximum(
            jnp.dot(x, awx_ref[i], preferred_element_type=jnp.float32)
            + jnp.dot(aggr, awa_ref[i], preferred_element_type=jnp.float32)
            + ab_ref[:, i, :], 0.0)
        out = out + ei * g[:, i:i + 1]
    nrm = jnp.maximum(jnp.sqrt(jnp.sum(out * out, axis=1, keepdims=True)),
                      1e-12)
    return jnp.maximum(out / nrm, 0.0)


def _moe_mid_body(x_ref, s0_ref, s1_ref, c0_ref, c1_ref, wgx_ref, wga_ref,
                  wgb_ref, awx_ref, awa_ref, ab_ref, lw_ref, lb_ref,
                  h_ref, y_ref):
    h = _moe_common(x_ref, s0_ref, s1_ref, c0_ref, c1_ref, wgx_ref, wga_ref,
                    wgb_ref, awx_ref, awa_ref, ab_ref)
    h_ref[...] = h
    y_ref[...] = jnp.maximum(
        jnp.dot(h, lw_ref[...], preferred_element_type=jnp.float32)
        + lb_ref[...], 0.0)


def _moe_fin_body(x_ref, s0_ref, s1_ref, c0_ref, c1_ref, wgx_ref, wga_ref,
                  wgb_ref, awx_ref, awa_ref, ab_ref, p1w_ref, p1b_ref,
                  p2w_ref, p2b_ref, o_ref):
    h = _moe_common(x_ref, s0_ref, s1_ref, c0_ref, c1_ref, wgx_ref, wga_ref,
                    wgb_ref, awx_ref, awa_ref, ab_ref)
    z = (jnp.dot(h, p1w_ref[...], preferred_element_type=jnp.float32)
         + p1b_ref[...])
    z = (jnp.dot(z, p2w_ref[...], preferred_element_type=jnp.float32)
         + p2b_ref[...])
    m = jnp.max(z, axis=1, keepdims=True)
    lse = m + jnp.log(jnp.sum(jnp.exp(z - m), axis=1, keepdims=True))
    o_ref[...] = z - lse


def _moe_specs(din, h, extra_specs):
    row = lambda w: pl.BlockSpec((BLK, w), lambda i: (i, 0))
    full = lambda *s: pl.BlockSpec(s, lambda i: (0,) * len(s))
    return [
        row(din), row(h), row(h), row(CNTW), row(CNTW),
        full(din, NEXP), full(h, NEXP), full(1, NEXP),
        full(NEXP, din, h), full(NEXP, h, h), full(1, NEXP, h),
    ] + extra_specs


def _moe_mid(x, s0, s1, c0, c1, wg_w, wg_b, agg_w, agg_b, lw, lb):
    n, din = x.shape
    h = s0.shape[1]
    full = lambda *s: pl.BlockSpec(s, lambda i: (0,) * len(s))
    return pl.pallas_call(
        _moe_mid_body,
        grid=(n // BLK,),
        in_specs=_moe_specs(din, h, [full(h, h), full(1, h)]),
        out_specs=[pl.BlockSpec((BLK, h), lambda i: (i, 0)),
                   pl.BlockSpec((BLK, h), lambda i: (i, 0))],
        out_shape=[jax.ShapeDtypeStruct((n, h), jnp.float32),
                   jax.ShapeDtypeStruct((n, h), jnp.float32)],
    )(x, s0, s1, c0, c1, wg_w[:din], wg_w[din:], wg_b.reshape(1, NEXP),
      agg_w[:, :din], agg_w[:, din:], agg_b.reshape(1, NEXP, h),
      lw, lb.reshape(1, h))


def _moe_fin(x, s0, s1, c0, c1, wg_w, wg_b, agg_w, agg_b, p1w, p1b, p2w, p2b):
    n, din = x.shape
    h = s0.shape[1]
    o = p2w.shape[1]
    full = lambda *s: pl.BlockSpec(s, lambda i: (0,) * len(s))
    return pl.pallas_call(
        _moe_fin_body,
        grid=(n // BLK,),
        in_specs=_moe_specs(din, h, [full(h, h), full(1, h),
                                     full(h, o), full(1, o)]),
        out_specs=pl.BlockSpec((BLK, o), lambda i: (i, 0)),
        out_shape=jax.ShapeDtypeStruct((n, o), jnp.float32),
    )(x, s0, s1, c0, c1, wg_w[:din], wg_w[din:], wg_b.reshape(1, NEXP),
      agg_w[:, :din], agg_w[:, din:], agg_b.reshape(1, NEXP, h),
      p1w, p1b.reshape(1, h), p2w, p2b.reshape(1, o))


def kernel(x, edge_index, lin_W0, lin_b0, Wg_W0, Wg_b0, agg_W0, agg_b0,
           lin_W1, lin_b1, Wg_W1, Wg_b1, agg_W1, agg_b1,
           p1_W, p1_b, p2_W, p2_b):
    src = edge_index[0]
    dst = edge_index[1]
    npad_edges = EPAD - src.shape[0]
    srcp = jnp.concatenate(
        [src, jnp.zeros((npad_edges,), jnp.int32)]).reshape(NCHUNKS, CHUNK)
    pad_dst = N + jnp.arange(npad_edges, dtype=jnp.int32) % (NPAD - N)
    dstp = jnp.concatenate([dst, pad_dst]).reshape(NCHUNKS, CHUNK)

    y0 = _prelin(x, lin_W0, lin_b0)
    c0a, c0b = _sc_cnt(dstp)
    s0a, s0b = _sc_segsum(y0, srcp, dstp)
    h1, y1 = _moe_mid(x, s0a, s0b, c0a, c0b, Wg_W0, Wg_b0, agg_W0, agg_b0,
                      lin_W1, lin_b1)
    s1a, s1b = _sc_segsum(y1, srcp, dstp)
    return _moe_fin(h1, s1a, s1b, c0a, c0b, Wg_W1, Wg_b1, agg_W1, agg_b1,
                    p1_W, p1_b, p2_W, p2_b)


# speedup vs baseline: 2.6015x; 2.2693x over previous
import functools

import jax
import jax.numpy as jnp
from jax import lax
from jax.experimental import pallas as pl
from jax.experimental.pallas import tpu as pltpu
from jax.experimental.pallas import tpu_sc as plsc

N = 10000
E = 320000
D = 128
NEXP = 4

NTILES = 32
CHUNK = 80
NSEG = 125
CCHUNK = 128
NCHUNKS = 2560
EPT = NCHUNKS // NTILES
NPAD = 10112
ROWS_PER_TILE = NPAD // 16
CNTW = 16
EPAD = NCHUNKS * CCHUNK


def _sc_segsum_body(y_hbm, src_hbm, dst_hbm, s0_out, s1_out,
                    src_v0, dst_v0, src_v1, dst_v1, rows_a, rows_b, sums_sh,
                    sem_a, sem_b):
    cid = lax.axis_index("c")
    sid = lax.axis_index("s")
    wid = cid * 16 + sid
    r0 = sid * ROWS_PER_TILE
    zrow = jnp.zeros((16,), jnp.float32)

    def fill_z(i, _):
        for c in range(8):
            rows_a[i, pl.ds(c * 16, 16)] = zrow
        return _

    lax.fori_loop(0, CHUNK, fill_z, None)
    for cp in range(ROWS_PER_TILE // CHUNK):
        pltpu.sync_copy(rows_a, sums_sh.at[pl.ds(r0 + cp * CHUNK, CHUNK)])
    pltpu.sync_copy(rows_a.at[pl.ds(0, ROWS_PER_TILE % CHUNK)],
                    sums_sh.at[pl.ds(r0 + (ROWS_PER_TILE // CHUNK) * CHUNK,
                                     ROWS_PER_TILE % CHUNK)])
    plsc.subcore_barrier()

    ebase = wid * (NSEG * CHUNK)

    def load_idx(sv, dv, b):
        pltpu.sync_copy(src_hbm.at[pl.ds(b, CHUNK)], sv)
        pltpu.sync_copy(dst_hbm.at[pl.ds(b, CHUNK)], dv)

    load_idx(src_v0, dst_v0, ebase)
    pltpu.async_copy(y_hbm.at[src_v0], rows_a, sem_a)

    def pair_body(p, _):
        load_idx(src_v1, dst_v1, ebase + (2 * p + 1) * CHUNK)
        pltpu.async_copy(y_hbm.at[src_v1], rows_b, sem_b)
        pltpu.make_async_copy(y_hbm.at[src_v0], rows_a, sem_a).wait()
        pltpu.sync_copy(rows_a, sums_sh.at[dst_v0], add=True)
        load_idx(src_v0, dst_v0, ebase + (2 * p + 2) * CHUNK)
        pltpu.async_copy(y_hbm.at[src_v0], rows_a, sem_a)
        pltpu.make_async_copy(y_hbm.at[src_v1], rows_b, sem_b).wait()
        pltpu.sync_copy(rows_b, sums_sh.at[dst_v1], add=True)
        return _

    lax.fori_loop(0, NSEG // 2, pair_body, None)
    pltpu.make_async_copy(y_hbm.at[src_v0], rows_a, sem_a).wait()
    pltpu.sync_copy(rows_a, sums_sh.at[dst_v0], add=True)
    plsc.subcore_barrier()

    @pl.when(cid == 0)
    def _():
        pltpu.sync_copy(sums_sh.at[pl.ds(r0, ROWS_PER_TILE)],
                        s0_out.at[pl.ds(r0, ROWS_PER_TILE)])

    @pl.when(cid == 1)
    def _():
        pltpu.sync_copy(sums_sh.at[pl.ds(r0, ROWS_PER_TILE)],
                        s1_out.at[pl.ds(r0, ROWS_PER_TILE)])


_sc_segsum = functools.partial(
    pl.kernel,
    out_type=[
        jax.ShapeDtypeStruct((NPAD, D), jnp.float32),
        jax.ShapeDtypeStruct((NPAD, D), jnp.float32),
    ],
    mesh=plsc.VectorSubcoreMesh(core_axis_name="c", subcore_axis_name="s"),
    scratch_types=[
        pltpu.VMEM((CHUNK,), jnp.int32),
        pltpu.VMEM((CHUNK,), jnp.int32),
        pltpu.VMEM((CHUNK,), jnp.int32),
        pltpu.VMEM((CHUNK,), jnp.int32),
        pltpu.VMEM((CHUNK, D), jnp.float32),
        pltpu.VMEM((CHUNK, D), jnp.float32),
        pltpu.VMEM_SHARED((NPAD, D), jnp.float32),
        pltpu.SemaphoreType.DMA,
        pltpu.SemaphoreType.DMA,
    ],
)(_sc_segsum_body)


def _sc_cnt_body(dst_hbm, c0_out, c1_out, dst_v, ones_v, zb16, cnt_sh, sem):
    cid = lax.axis_index("c")
    sid = lax.axis_index("s")
    wid = cid * 16 + sid
    r0 = sid * ROWS_PER_TILE
    zrow = jnp.zeros((16,), jnp.float32)

    pltpu.sync_copy(dst_hbm.at[pl.ds(wid * EPT, EPT)], dst_v)

    def fill_ones(i, _):
        ones_v[i, :] = zrow + 1.0
        zb16[i, :] = zrow
        return _

    lax.fori_loop(0, CCHUNK, fill_ones, None)
    for cp in range(ROWS_PER_TILE // CCHUNK):
        pltpu.sync_copy(zb16, cnt_sh.at[pl.ds(r0 + cp * CCHUNK, CCHUNK)])
    pltpu.sync_copy(zb16.at[pl.ds(0, ROWS_PER_TILE % CCHUNK)],
                    cnt_sh.at[pl.ds(r0 + (ROWS_PER_TILE // CCHUNK) * CCHUNK,
                                    ROWS_PER_TILE % CCHUNK)])
    plsc.subcore_barrier()

    def chunk_body(j, _):
        pltpu.sync_copy(ones_v, cnt_sh.at[dst_v.at[j]], add=True)
        return _

    lax.fori_loop(0, EPT, chunk_body, None)
    plsc.subcore_barrier()

    @pl.when(cid == 0)
    def _():
        pltpu.sync_copy(cnt_sh.at[pl.ds(r0, ROWS_PER_TILE)],
                        c0_out.at[pl.ds(r0, ROWS_PER_TILE)])

    @pl.when(cid == 1)
    def _():
        pltpu.sync_copy(cnt_sh.at[pl.ds(r0, ROWS_PER_TILE)],
                        c1_out.at[pl.ds(r0, ROWS_PER_TILE)])


_sc_cnt = functools.partial(
    pl.kernel,
    out_type=[
        jax.ShapeDtypeStruct((NPAD, CNTW), jnp.float32),
        jax.ShapeDtypeStruct((NPAD, CNTW), jnp.float32),
    ],
    mesh=plsc.VectorSubcoreMesh(core_axis_name="c", subcore_axis_name="s"),
    scratch_types=[
        pltpu.VMEM((EPT, CCHUNK), jnp.int32),
        pltpu.VMEM((CCHUNK, CNTW), jnp.float32),
        pltpu.VMEM((CCHUNK, CNTW), jnp.float32),
        pltpu.VMEM_SHARED((NPAD, CNTW), jnp.float32),
        pltpu.SemaphoreType.DMA,
    ],
)(_sc_cnt_body)


BLK = 1000


def _prelin_body(x_ref, w_ref, b_ref, o_ref):
    o_ref[...] = jnp.maximum(
        jnp.dot(x_ref[...], w_ref[...], preferred_element_type=jnp.float32)
        + b_ref[...], 0.0)


def _prelin(x, w, b):
    n, d = x.shape
    h = w.shape[1]
    return pl.pallas_call(
        _prelin_body,
        grid=(n // BLK,),
        in_specs=[
            pl.BlockSpec((BLK, d), lambda i: (i, 0)),
            pl.BlockSpec((d, h), lambda i: (0, 0)),
            pl.BlockSpec((1, h), lambda i: (0, 0)),
        ],
        out_specs=pl.BlockSpec((BLK, h), lambda i: (i, 0)),
        out_shape=jax.ShapeDtypeStruct((n, h), jnp.float32),
    )(x, w, b.reshape(1, h))


def _moe_common(x_ref, s0_ref, s1_ref, c0_ref, c1_ref, wgx_ref, wga_ref,
                wgb_ref, awx_ref, awa_ref, ab_ref):
    x = x_ref[...]
    cnt = c0_ref[:, 0:1] + c1_ref[:, 0:1]
    aggr = (s0_ref[...] + s1_ref[...]) / jnp.maximum(cnt, 1.0)
    logits = (jnp.dot(x, wgx_ref[...], preferred_element_type=jnp.float32)
              + jnp.dot(aggr, wga_ref[...], preferred_element_type=jnp.float32)
              + wgb_ref[...])
    m = jnp.max(logits, axis=1, keepdims=True)
    eg = jnp.exp(logits - m)
    g = eg / jnp.sum(eg, axis=1, keepdims=True)
    out = jnp.zeros_like(aggr)
    for i in range(NEXP):
        ei = jnp.maximum(
            jnp.dot(x, awx_ref[i], preferred_element_type=jnp.float32)
            + jnp.dot(aggr, awa_ref[i], preferred_element_type=jnp.float32)
            + ab_ref[:, i, :], 0.0)
        out = out + ei * g[:, i:i + 1]
    nrm = jnp.maximum(jnp.sqrt(jnp.sum(out * out, axis=1, keepdims=True)),
                      1e-12)
    return jnp.maximum(out / nrm, 0.0)


def _moe_mid_body(x_ref, s0_ref, s1_ref, c0_ref, c1_ref, wgx_ref, wga_ref,
                  wgb_ref, awx_ref, awa_ref, ab_ref, lw_ref, lb_ref,
                  h_ref, y_ref):
    h = _moe_common(x_ref, s0_ref, s1_ref, c0_ref, c1_ref, wgx_ref, wga_ref,
                    wgb_ref, awx_ref, awa_ref, ab_ref)
    h_ref[...] = h
    y_ref[...] = jnp.maximum(
        jnp.dot(h, lw_ref[...], preferred_element_type=jnp.float32)
        + lb_ref[...], 0.0)


def _moe_fin_body(x_ref, s0_ref, s1_ref, c0_ref, c1_ref, wgx_ref, wga_ref,
                  wgb_ref, awx_ref, awa_ref, ab_ref, p1w_ref, p1b_ref,
                  p2w_ref, p2b_ref, o_ref):
    h = _moe_common(x_ref, s0_ref, s1_ref, c0_ref, c1_ref, wgx_ref, wga_ref,
                    wgb_ref, awx_ref, awa_ref, ab_ref)
    z = (jnp.dot(h, p1w_ref[...], preferred_element_type=jnp.float32)
         + p1b_ref[...])
    z = (jnp.dot(z, p2w_ref[...], preferred_element_type=jnp.float32)
         + p2b_ref[...])
    m = jnp.max(z, axis=1, keepdims=True)
    lse = m + jnp.log(jnp.sum(jnp.exp(z - m), axis=1, keepdims=True))
    o_ref[...] = z - lse


def _moe_specs(din, h, extra_specs):
    row = lambda w: pl.BlockSpec((BLK, w), lambda i: (i, 0))
    full = lambda *s: pl.BlockSpec(s, lambda i: (0,) * len(s))
    return [
        row(din), row(h), row(h), row(CNTW), row(CNTW),
        full(din, NEXP), full(h, NEXP), full(1, NEXP),
        full(NEXP, din, h), full(NEXP, h, h), full(1, NEXP, h),
    ] + extra_specs


def _moe_mid(x, s0, s1, c0, c1, wg_w, wg_b, agg_w, agg_b, lw, lb):
    n, din = x.shape
    h = s0.shape[1]
    full = lambda *s: pl.BlockSpec(s, lambda i: (0,) * len(s))
    return pl.pallas_call(
        _moe_mid_body,
        grid=(n // BLK,),
        in_specs=_moe_specs(din, h, [full(h, h), full(1, h)]),
        out_specs=[pl.BlockSpec((BLK, h), lambda i: (i, 0)),
                   pl.BlockSpec((BLK, h), lambda i: (i, 0))],
        out_shape=[jax.ShapeDtypeStruct((n, h), jnp.float32),
                   jax.ShapeDtypeStruct((n, h), jnp.float32)],
    )(x, s0, s1, c0, c1, wg_w[:din], wg_w[din:], wg_b.reshape(1, NEXP),
      agg_w[:, :din], agg_w[:, din:], agg_b.reshape(1, NEXP, h),
      lw, lb.reshape(1, h))


def _moe_fin(x, s0, s1, c0, c1, wg_w, wg_b, agg_w, agg_b, p1w, p1b, p2w, p2b):
    n, din = x.shape
    h = s0.shape[1]
    o = p2w.shape[1]
    full = lambda *s: pl.BlockSpec(s, lambda i: (0,) * len(s))
    return pl.pallas_call(
        _moe_fin_body,
        grid=(n // BLK,),
        in_specs=_moe_specs(din, h, [full(h, h), full(1, h),
                                     full(h, o), full(1, o)]),
        out_specs=pl.BlockSpec((BLK, o), lambda i: (i, 0)),
        out_shape=jax.ShapeDtypeStruct((n, o), jnp.float32),
    )(x, s0, s1, c0, c1, wg_w[:din], wg_w[din:], wg_b.reshape(1, NEXP),
      agg_w[:, :din], agg_w[:, din:], agg_b.reshape(1, NEXP, h),
      p1w, p1b.reshape(1, h), p2w, p2b.reshape(1, o))


def kernel(x, edge_index, lin_W0, lin_b0, Wg_W0, Wg_b0, agg_W0, agg_b0,
           lin_W1, lin_b1, Wg_W1, Wg_b1, agg_W1, agg_b1,
           p1_W, p1_b, p2_W, p2_b):
    src = edge_index[0]
    dst = edge_index[1]
    npad_edges = EPAD - src.shape[0]
    pad_dst = N + jnp.arange(npad_edges, dtype=jnp.int32) % (NPAD - N)
    dstp = jnp.concatenate([dst, pad_dst]).reshape(NCHUNKS, CCHUNK)

    y0 = _prelin(x, lin_W0, lin_b0)
    c0a, c0b = _sc_cnt(dstp)
    s0a, s0b = _sc_segsum(y0, src, dst)
    h1, y1 = _moe_mid(x, s0a, s0b, c0a, c0b, Wg_W0, Wg_b0, agg_W0, agg_b0,
                      lin_W1, lin_b1)
    s1a, s1b = _sc_segsum(y1, src, dst)
    return _moe_fin(h1, s1a, s1b, c0a, c0b, Wg_W1, Wg_b1, agg_W1, agg_b1,
                    p1_W, p1_b, p2_W, p2_b)
